# baseline (speedup 1.0000x reference)
import functools

import jax
import jax.numpy as jnp
from jax import lax
from jax.experimental import pallas as pl
from jax.experimental.pallas import tpu as pltpu
from jax.experimental.pallas import tpu_sc as plsc

N = 10000
E = 320000
D_IN = 128
D_HID = 128
D_OUT = 40
GL1 = 0.5
GL2 = 0.05

NC, NS = 2, 16
NW = NC * NS
NP = 10240
RPT = NP // NS
D2P = 128
EC = 128
CPW = 80
EPW = CPW * EC
EP = EPW * NW
DC = 2 * CPW
CPW0, CPW1 = 160, 0
DC0, DC1 = 192, 128

_MESH = plsc.VectorSubcoreMesh(
    core_axis_name="c", subcore_axis_name="s", num_cores=NC, num_subcores=NS)


def _worker_ids():
    c = lax.axis_index("c")
    s = lax.axis_index("s")
    return c, s, s * NC + c


def _deg_body(allidx_hbm, deg_hbm, idx_v, ones_v, buf_v, deg_sh):
    c, s, wid = _worker_ids()
    for i in range(EC // 16):
        ones_v[pl.ds(i * 16, 16)] = jnp.ones((16,), jnp.float32)
    for i in range(RPT // 16):
        buf_v[pl.ds(i * 16, 16)] = jnp.zeros((16,), jnp.float32)

    pltpu.sync_copy(buf_v, deg_sh.at[pl.ds(s * RPT, RPT)])
    base = jnp.where(c == 0, s * DC0, NS * DC0 + s * DC1)
    dc = jnp.where(c == 0, DC0, DC1)

    @pl.when(c == 0)
    def _():
        pltpu.sync_copy(allidx_hbm.at[pl.ds(base, DC0)], idx_v)

    @pl.when(c != 0)
    def _():
        pltpu.sync_copy(allidx_hbm.at[pl.ds(base, DC1)],
                        idx_v.at[pl.ds(0, DC1)])

    plsc.subcore_barrier()

    def chunk(j, carry):
        pltpu.sync_copy(ones_v, deg_sh.at[idx_v.at[j]], add=True)
        return carry

    lax.fori_loop(0, dc, chunk, 0)
    plsc.subcore_barrier()
    pltpu.sync_copy(deg_sh.at[pl.ds(s * RPT, RPT)], buf_v)
    pltpu.sync_copy(buf_v, deg_hbm.at[pl.ds(c * NP + s * RPT, RPT)])


_deg_kernel = pl.kernel(
    _deg_body,
    out_type=jax.ShapeDtypeStruct((NC * NP,), jnp.float32),
    mesh=_MESH,
    scratch_types=[
        pltpu.VMEM((DC0, EC), jnp.int32),
        pltpu.VMEM((EC,), jnp.float32),
        pltpu.VMEM((RPT,), jnp.float32),
        pltpu.VMEM_SHARED((NP,), jnp.float32),
    ],
)


IB = 8
NB0 = CPW0 // IB
NB1 = CPW1 // IB


def _msg_body(d, g_hbm, src_hbm, dst_hbm, agg_hbm, srcv, dstv,
              rows_a, rows_b, sem_a, sem_b, agg_sh):
    c, s, wid = _worker_ids()
    base = jnp.where(c == 0, s * CPW0, NS * CPW0 + s * CPW1)
    nb = jnp.where(c == 0, NB0, NB1)

    def zrow(r, carry):
        for k in range(d // 16):
            rows_a[r, pl.ds(k * 16, 16)] = jnp.zeros((16,), jnp.float32)
        return carry

    lax.fori_loop(0, EC, zrow, 0)

    for t in range(RPT // EC):
        pltpu.sync_copy(rows_a, agg_sh.at[pl.ds(s * RPT + t * EC, EC)])
    @pl.when(nb > 0)
    def _():
        pltpu.sync_copy(src_hbm.at[pl.ds(base, IB)], srcv.at[0])
        pltpu.sync_copy(dst_hbm.at[pl.ds(base, IB)], dstv.at[0])

    plsc.subcore_barrier()

    @pl.when(nb > 0)
    def _():
        pltpu.async_copy(g_hbm.at[srcv.at[0, 0]], rows_a, sem_a)

    def block_body(b, carry):
        sb = lax.rem(b, 2)
        nsb = 1 - sb

        @pl.when(b < nb - 1)
        def _():
            pltpu.sync_copy(src_hbm.at[pl.ds(base + (b + 1) * IB, IB)],
                            srcv.at[nsb])
            pltpu.sync_copy(dst_hbm.at[pl.ds(base + (b + 1) * IB, IB)],
                            dstv.at[nsb])

        for jj in range(IB):
            if jj % 2 == 0:
                buf, sem, obuf, osem = rows_a, sem_a, rows_b, sem_b
            else:
                buf, sem, obuf, osem = rows_b, sem_b, rows_a, sem_a
            if jj < IB - 1:
                pltpu.async_copy(g_hbm.at[srcv.at[sb, jj + 1]], obuf, osem)
            else:
                @pl.when(b < nb - 1)
                def _(nsb=nsb, obuf=obuf, osem=osem):
                    pltpu.async_copy(g_hbm.at[srcv.at[nsb, 0]], obuf, osem)
            pltpu.make_async_copy(g_hbm.at[srcv.at[sb, jj]], buf, sem).wait()
            pltpu.sync_copy(buf, agg_sh.at[dstv.at[sb, jj]], add=True)
        return carry

    lax.fori_loop(0, nb, block_body, 0)
    plsc.subcore_barrier()
    for t in range(RPT // EC):
        tbase = s * RPT + t * EC
        pltpu.sync_copy(agg_sh.at[pl.ds(tbase, EC)], rows_a)
        pltpu.sync_copy(rows_a, agg_hbm.at[c, pl.ds(tbase, EC)])


def _make_msg_kernel(d):
    return pl.kernel(
        functools.partial(_msg_body, d),
        out_type=jax.ShapeDtypeStruct((NC, NP, d), jnp.float32),
        mesh=_MESH,
        scratch_types=[
            pltpu.VMEM((2, IB, EC), jnp.int32),
            pltpu.VMEM((2, IB, EC), jnp.int32),
            pltpu.VMEM((EC, d), jnp.float32),
            pltpu.VMEM((EC, d), jnp.float32),
            pltpu.SemaphoreType.DMA,
            pltpu.SemaphoreType.DMA,
            pltpu.VMEM_SHARED((NP, d), jnp.float32),
        ],
    )


_msg_kernel_128 = _make_msg_kernel(D_HID)


BR = 256
GRID = NP // BR


def _dinv_from(deg_blk):
    deg = deg_blk[0, :] + deg_blk[1, :]
    return 1.0 / jnp.sqrt(jnp.maximum(deg, 1.0))


def _stage_a_body(x_ref, w_ref, deg_ref, h0_ref, g0_ref, gram_ref):
    h0 = jnp.dot(x_ref[...], w_ref[...], preferred_element_type=jnp.float32)
    dinv = _dinv_from(deg_ref[...])
    h0_ref[...] = h0
    g0_ref[...] = h0 * dinv[:, None]

    @pl.when(pl.program_id(0) == 0)
    def _():
        gram_ref[...] = jnp.zeros_like(gram_ref)

    gram_ref[...] += lax.dot_general(
        h0, h0, (((0,), (0,)), ((), ())), preferred_element_type=jnp.float32)


_stage_a = pl.pallas_call(
    _stage_a_body,
    grid=(GRID,),
    in_specs=[
        pl.BlockSpec((BR, D_IN), lambda i: (i, 0)),
        pl.BlockSpec((D_IN, D_HID), lambda i: (0, 0)),
        pl.BlockSpec((NC, BR), lambda i: (0, i)),
    ],
    out_specs=[
        pl.BlockSpec((BR, D_HID), lambda i: (i, 0)),
        pl.BlockSpec((BR, D_HID), lambda i: (i, 0)),
        pl.BlockSpec((D_HID, D_HID), lambda i: (0, 0)),
    ],
    out_shape=[
        jax.ShapeDtypeStruct((NP, D_HID), jnp.float32),
        jax.ShapeDtypeStruct((NP, D_HID), jnp.float32),
        jax.ShapeDtypeStruct((D_HID, D_HID), jnp.float32),
    ],
)


def _stage_b_body(h0_ref, agg_ref, deg_ref, gram_ref, w2_ref,
                  h1_ref, g1_ref, gram1_ref):
    h0 = h0_ref[...]
    dinv = _dinv_from(deg_ref[...])
    aggs = (agg_ref[0] + agg_ref[1]) * dinv[:, None]
    eye = (lax.broadcasted_iota(jnp.int32, (D_HID, D_HID), 0)
           == lax.broadcasted_iota(jnp.int32, (D_HID, D_HID), 1))
    G = gram_ref[...] - eye.astype(jnp.float32)
    out1 = ((1.0 - GL1) * h0 + GL1 * aggs
            - GL2 * jnp.dot(h0, G, preferred_element_type=jnp.float32))
    h1 = jnp.dot(jnp.maximum(out1, 0.0), w2_ref[...],
                 preferred_element_type=jnp.float32)
    h1_ref[...] = h1
    g1_ref[...] = h1 * dinv[:, None]

    @pl.when(pl.program_id(0) == 0)
    def _():
        gram1_ref[...] = jnp.zeros_like(gram1_ref)

    gram1_ref[...] += lax.dot_general(
        h1, h1, (((0,), (0,)), ((), ())), preferred_element_type=jnp.float32)


_stage_b = pl.pallas_call(
    _stage_b_body,
    grid=(GRID,),
    in_specs=[
        pl.BlockSpec((BR, D_HID), lambda i: (i, 0)),
        pl.BlockSpec((NC, BR, D_HID), lambda i: (0, i, 0)),
        pl.BlockSpec((NC, BR), lambda i: (0, i)),
        pl.BlockSpec((D_HID, D_HID), lambda i: (0, 0)),
        pl.BlockSpec((D_HID, D2P), lambda i: (0, 0)),
    ],
    out_specs=[
        pl.BlockSpec((BR, D2P), lambda i: (i, 0)),
        pl.BlockSpec((BR, D2P), lambda i: (i, 0)),
        pl.BlockSpec((D2P, D2P), lambda i: (0, 0)),
    ],
    out_shape=[
        jax.ShapeDtypeStruct((NP, D2P), jnp.float32),
        jax.ShapeDtypeStruct((NP, D2P), jnp.float32),
        jax.ShapeDtypeStruct((D2P, D2P), jnp.float32),
    ],
)


def _stage_c_body(h1_ref, agg_ref, deg_ref, gram1_ref, out_ref):
    h1 = h1_ref[...]
    dinv = _dinv_from(deg_ref[...])
    aggs = (agg_ref[0] + agg_ref[1]) * dinv[:, None]
    eye = (lax.broadcasted_iota(jnp.int32, (D2P, D2P), 0)
           == lax.broadcasted_iota(jnp.int32, (D2P, D2P), 1))
    G = gram1_ref[...] - eye.astype(jnp.float32)
    out2 = ((1.0 - GL1) * h1 + GL1 * aggs
            - GL2 * jnp.dot(h1, G, preferred_element_type=jnp.float32))
    valid = lax.broadcasted_iota(jnp.int32, (BR, D2P), 1) < D_OUT
    masked = jnp.where(valid, out2, -jnp.inf)
    m = jnp.max(masked, axis=1, keepdims=True)
    ex = jnp.where(valid, jnp.exp(out2 - m), 0.0)
    lse = jnp.log(jnp.sum(ex, axis=1, keepdims=True))
    out_ref[...] = out2 - m - lse


_stage_c = pl.pallas_call(
    _stage_c_body,
    grid=(GRID,),
    in_specs=[
        pl.BlockSpec((BR, D2P), lambda i: (i, 0)),
        pl.BlockSpec((NC, BR, D2P), lambda i: (0, i, 0)),
        pl.BlockSpec((NC, BR), lambda i: (0, i)),
        pl.BlockSpec((D2P, D2P), lambda i: (0, 0)),
    ],
    out_specs=pl.BlockSpec((BR, D2P), lambda i: (i, 0)),
    out_shape=jax.ShapeDtypeStruct((NP, D2P), jnp.float32),
)


def kernel(x, y, edge_index, W_in, W_out):
    del y
    pad = jnp.full((EP - E,), NP - 1, jnp.int32)
    srcp = jnp.concatenate([edge_index[0], pad])
    dstp = jnp.concatenate([edge_index[1], pad])
    allidx = jnp.concatenate([srcp, dstp]).reshape(NW * DC, EC)
    src2 = srcp.reshape(NW * CPW, EC)
    dst2 = dstp.reshape(NW * CPW, EC)
    x_pad = jnp.pad(x, ((0, NP - N), (0, 0)))
    w2_pad = jnp.pad(W_out, ((0, 0), (0, D2P - D_OUT)))

    deg = _deg_kernel(allidx).reshape(NC, NP)
    h0, g0, gram0 = _stage_a(x_pad, W_in, deg)
    agg1 = _msg_kernel_128(g0, src2, dst2)
    h1, g1, gram1 = _stage_b(h0, agg1, deg, gram0, w2_pad)
    agg2 = _msg_kernel_128(g1, src2, dst2)
    outp = _stage_c(h1, agg2, deg, gram1)
    return outp[:N, :D_OUT]


# speedup vs baseline: 2.7374x; 2.7374x over previous
import functools

import jax
import jax.numpy as jnp
from jax import lax
from jax.experimental import pallas as pl
from jax.experimental.pallas import tpu as pltpu
from jax.experimental.pallas import tpu_sc as plsc

N = 10000
E = 320000
D_IN = 128
D_HID = 128
D_OUT = 40
GL1 = 0.5
GL2 = 0.05

NC, NS = 2, 16
NW = NC * NS
NP = 10240
RPT = NP // NS
D2P = 128
EC = 128
CPW = 80
EPW = CPW * EC
EP = EPW * NW
DC = 2 * CPW
CPW0, CPW1 = 80, 80
DC0, DC1 = 160, 160

_MESH = plsc.VectorSubcoreMesh(
    core_axis_name="c", subcore_axis_name="s", num_cores=NC, num_subcores=NS)


def _worker_ids():
    c = lax.axis_index("c")
    s = lax.axis_index("s")
    return c, s, s * NC + c


def _deg_body(allidx_hbm, deg_hbm, idx_v, ones_v, buf_v, deg_sh):
    c, s, wid = _worker_ids()
    for i in range(EC // 16):
        ones_v[pl.ds(i * 16, 16)] = jnp.ones((16,), jnp.float32)
    for i in range(RPT // 16):
        buf_v[pl.ds(i * 16, 16)] = jnp.zeros((16,), jnp.float32)

    pltpu.sync_copy(buf_v, deg_sh.at[pl.ds(s * RPT, RPT)])
    base = jnp.where(c == 0, s * DC0, NS * DC0 + s * DC1)
    dc = jnp.where(c == 0, DC0, DC1)

    @pl.when(c == 0)
    def _():
        pltpu.sync_copy(allidx_hbm.at[pl.ds(base, DC0)], idx_v)

    @pl.when(c != 0)
    def _():
        pltpu.sync_copy(allidx_hbm.at[pl.ds(base, DC1)],
                        idx_v.at[pl.ds(0, DC1)])

    plsc.subcore_barrier()

    def chunk(j, carry):
        pltpu.sync_copy(ones_v, deg_sh.at[idx_v.at[j]], add=True)
        return carry

    lax.fori_loop(0, dc, chunk, 0)
    plsc.subcore_barrier()
    pltpu.sync_copy(deg_sh.at[pl.ds(s * RPT, RPT)], buf_v)
    pltpu.sync_copy(buf_v, deg_hbm.at[pl.ds(c * NP + s * RPT, RPT)])


_deg_kernel = pl.kernel(
    _deg_body,
    out_type=jax.ShapeDtypeStruct((NC * NP,), jnp.float32),
    mesh=_MESH,
    scratch_types=[
        pltpu.VMEM((DC0, EC), jnp.int32),
        pltpu.VMEM((EC,), jnp.float32),
        pltpu.VMEM((RPT,), jnp.float32),
        pltpu.VMEM_SHARED((NP,), jnp.float32),
    ],
)


IB = 8
NB0 = CPW0 // IB
NB1 = CPW1 // IB


def _msg_body(d, g_hbm, src_hbm, dst_hbm, agg_hbm, srcv, dstv,
              rows_a, rows_b, sem_a, sem_b, agg_sh):
    c, s, wid = _worker_ids()
    base = jnp.where(c == 0, s * CPW0, NS * CPW0 + s * CPW1)
    nb = jnp.where(c == 0, NB0, NB1)

    def zrow(r, carry):
        for k in range(d // 16):
            rows_a[r, pl.ds(k * 16, 16)] = jnp.zeros((16,), jnp.float32)
        return carry

    lax.fori_loop(0, EC, zrow, 0)

    for t in range(RPT // EC):
        pltpu.sync_copy(rows_a, agg_sh.at[pl.ds(s * RPT + t * EC, EC)])
    @pl.when(nb > 0)
    def _():
        pltpu.sync_copy(src_hbm.at[pl.ds(base, IB)], srcv.at[0])
        pltpu.sync_copy(dst_hbm.at[pl.ds(base, IB)], dstv.at[0])

    plsc.subcore_barrier()

    @pl.when(nb > 0)
    def _():
        pltpu.async_copy(g_hbm.at[srcv.at[0, 0]], rows_a, sem_a)

    def block_body(b, carry):
        sb = lax.rem(b, 2)
        nsb = 1 - sb

        @pl.when(b < nb - 1)
        def _():
            pltpu.sync_copy(src_hbm.at[pl.ds(base + (b + 1) * IB, IB)],
                            srcv.at[nsb])
            pltpu.sync_copy(dst_hbm.at[pl.ds(base + (b + 1) * IB, IB)],
                            dstv.at[nsb])

        for jj in range(IB):
            if jj % 2 == 0:
                buf, sem, obuf, osem = rows_a, sem_a, rows_b, sem_b
            else:
                buf, sem, obuf, osem = rows_b, sem_b, rows_a, sem_a
            if jj < IB - 1:
                pltpu.async_copy(g_hbm.at[srcv.at[sb, jj + 1]], obuf, osem)
            else:
                @pl.when(b < nb - 1)
                def _(nsb=nsb, obuf=obuf, osem=osem):
                    pltpu.async_copy(g_hbm.at[srcv.at[nsb, 0]], obuf, osem)
            pltpu.make_async_copy(g_hbm.at[srcv.at[sb, jj]], buf, sem).wait()
            pltpu.sync_copy(buf, agg_sh.at[dstv.at[sb, jj]], add=True)
        return carry

    lax.fori_loop(0, nb, block_body, 0)
    plsc.subcore_barrier()
    for t in range(RPT // EC):
        tbase = s * RPT + t * EC
        pltpu.sync_copy(agg_sh.at[pl.ds(tbase, EC)], rows_a)
        pltpu.sync_copy(rows_a, agg_hbm.at[c, pl.ds(tbase, EC)])


def _make_msg_kernel(d):
    return pl.kernel(
        functools.partial(_msg_body, d),
        out_type=jax.ShapeDtypeStruct((NC, NP, d), jnp.float32),
        mesh=_MESH,
        scratch_types=[
            pltpu.VMEM((2, IB, EC), jnp.int32),
            pltpu.VMEM((2, IB, EC), jnp.int32),
            pltpu.VMEM((EC, d), jnp.float32),
            pltpu.VMEM((EC, d), jnp.float32),
            pltpu.SemaphoreType.DMA,
            pltpu.SemaphoreType.DMA,
            pltpu.VMEM_SHARED((NP, d), jnp.float32),
        ],
    )


_msg_kernel_128 = _make_msg_kernel(D_HID)


BR = 256
GRID = NP // BR


def _dinv_from(deg_blk):
    deg = deg_blk[0, :] + deg_blk[1, :]
    return 1.0 / jnp.sqrt(jnp.maximum(deg, 1.0))


def _stage_a_body(x_ref, w_ref, deg_ref, h0_ref, g0_ref, gram_ref):
    h0 = jnp.dot(x_ref[...], w_ref[...], preferred_element_type=jnp.float32)
    dinv = _dinv_from(deg_ref[...])
    h0_ref[...] = h0
    g0_ref[...] = h0 * dinv[:, None]

    @pl.when(pl.program_id(0) == 0)
    def _():
        gram_ref[...] = jnp.zeros_like(gram_ref)

    gram_ref[...] += lax.dot_general(
        h0, h0, (((0,), (0,)), ((), ())), preferred_element_type=jnp.float32)


_stage_a = pl.pallas_call(
    _stage_a_body,
    grid=(GRID,),
    in_specs=[
        pl.BlockSpec((BR, D_IN), lambda i: (i, 0)),
        pl.BlockSpec((D_IN, D_HID), lambda i: (0, 0)),
        pl.BlockSpec((NC, BR), lambda i: (0, i)),
    ],
    out_specs=[
        pl.BlockSpec((BR, D_HID), lambda i: (i, 0)),
        pl.BlockSpec((BR, D_HID), lambda i: (i, 0)),
        pl.BlockSpec((D_HID, D_HID), lambda i: (0, 0)),
    ],
    out_shape=[
        jax.ShapeDtypeStruct((NP, D_HID), jnp.float32),
        jax.ShapeDtypeStruct((NP, D_HID), jnp.float32),
        jax.ShapeDtypeStruct((D_HID, D_HID), jnp.float32),
    ],
)


def _stage_b_body(h0_ref, agg_ref, deg_ref, gram_ref, w2_ref,
                  h1_ref, g1_ref, gram1_ref):
    h0 = h0_ref[...]
    dinv = _dinv_from(deg_ref[...])
    aggs = (agg_ref[0] + agg_ref[1]) * dinv[:, None]
    eye = (lax.broadcasted_iota(jnp.int32, (D_HID, D_HID), 0)
           == lax.broadcasted_iota(jnp.int32, (D_HID, D_HID), 1))
    G = gram_ref[...] - eye.astype(jnp.float32)
    out1 = ((1.0 - GL1) * h0 + GL1 * aggs
            - GL2 * jnp.dot(h0, G, preferred_element_type=jnp.float32))
    h1 = jnp.dot(jnp.maximum(out1, 0.0), w2_ref[...],
                 preferred_element_type=jnp.float32)
    h1_ref[...] = h1
    g1_ref[...] = h1 * dinv[:, None]

    @pl.when(pl.program_id(0) == 0)
    def _():
        gram1_ref[...] = jnp.zeros_like(gram1_ref)

    gram1_ref[...] += lax.dot_general(
        h1, h1, (((0,), (0,)), ((), ())), preferred_element_type=jnp.float32)


_stage_b = pl.pallas_call(
    _stage_b_body,
    grid=(GRID,),
    in_specs=[
        pl.BlockSpec((BR, D_HID), lambda i: (i, 0)),
        pl.BlockSpec((NC, BR, D_HID), lambda i: (0, i, 0)),
        pl.BlockSpec((NC, BR), lambda i: (0, i)),
        pl.BlockSpec((D_HID, D_HID), lambda i: (0, 0)),
        pl.BlockSpec((D_HID, D2P), lambda i: (0, 0)),
    ],
    out_specs=[
        pl.BlockSpec((BR, D2P), lambda i: (i, 0)),
        pl.BlockSpec((BR, D2P), lambda i: (i, 0)),
        pl.BlockSpec((D2P, D2P), lambda i: (0, 0)),
    ],
    out_shape=[
        jax.ShapeDtypeStruct((NP, D2P), jnp.float32),
        jax.ShapeDtypeStruct((NP, D2P), jnp.float32),
        jax.ShapeDtypeStruct((D2P, D2P), jnp.float32),
    ],
)


def _stage_c_body(h1_ref, agg_ref, deg_ref, gram1_ref, out_ref):
    h1 = h1_ref[...]
    dinv = _dinv_from(deg_ref[...])
    aggs = (agg_ref[0] + agg_ref[1]) * dinv[:, None]
    eye = (lax.broadcasted_iota(jnp.int32, (D2P, D2P), 0)
           == lax.broadcasted_iota(jnp.int32, (D2P, D2P), 1))
    G = gram1_ref[...] - eye.astype(jnp.float32)
    out2 = ((1.0 - GL1) * h1 + GL1 * aggs
            - GL2 * jnp.dot(h1, G, preferred_element_type=jnp.float32))
    valid = lax.broadcasted_iota(jnp.int32, (BR, D2P), 1) < D_OUT
    masked = jnp.where(valid, out2, -jnp.inf)
    m = jnp.max(masked, axis=1, keepdims=True)
    ex = jnp.where(valid, jnp.exp(out2 - m), 0.0)
    lse = jnp.log(jnp.sum(ex, axis=1, keepdims=True))
    out_ref[...] = out2 - m - lse


_stage_c = pl.pallas_call(
    _stage_c_body,
    grid=(GRID,),
    in_specs=[
        pl.BlockSpec((BR, D2P), lambda i: (i, 0)),
        pl.BlockSpec((NC, BR, D2P), lambda i: (0, i, 0)),
        pl.BlockSpec((NC, BR), lambda i: (0, i)),
        pl.BlockSpec((D2P, D2P), lambda i: (0, 0)),
    ],
    out_specs=pl.BlockSpec((BR, D2P), lambda i: (i, 0)),
    out_shape=jax.ShapeDtypeStruct((NP, D2P), jnp.float32),
)


def kernel(x, y, edge_index, W_in, W_out):
    del y
    pad = N + (jnp.arange(EP - E, dtype=jnp.int32) % (NP - N))
    srcp = jnp.concatenate([edge_index[0], pad])
    dstp = jnp.concatenate([edge_index[1], pad])
    allidx = jnp.concatenate([srcp, dstp]).reshape(NW * DC, EC)
    src2 = srcp.reshape(NW * CPW, EC)
    dst2 = dstp.reshape(NW * CPW, EC)
    x_pad = jnp.pad(x, ((0, NP - N), (0, 0)))
    w2_pad = jnp.pad(W_out, ((0, 0), (0, D2P - D_OUT)))

    deg = _deg_kernel(allidx).reshape(NC, NP)
    h0, g0, gram0 = _stage_a(x_pad, W_in, deg)
    agg1 = _msg_kernel_128(g0, src2, dst2)
    h1, g1, gram1 = _stage_b(h0, agg1, deg, gram0, w2_pad)
    agg2 = _msg_kernel_128(g1, src2, dst2)
    outp = _stage_c(h1, agg2, deg, gram1)
    return outp[:N, :D_OUT]


# speedup vs baseline: 2.7682x; 1.0113x over previous
import functools

import jax
import jax.numpy as jnp
from jax import lax
from jax.experimental import pallas as pl
from jax.experimental.pallas import tpu as pltpu
from jax.experimental.pallas import tpu_sc as plsc

N = 10000
E = 320000
D_IN = 128
D_HID = 128
D_OUT = 40
GL1 = 0.5
GL2 = 0.05

NC, NS = 2, 16
NW = NC * NS
NP = 10240
RPT = NP // NS
D2P = 128
EC = 128
CPW = 80
EPW = CPW * EC
EP = EPW * NW
DC = 2 * CPW
CPW0, CPW1 = 80, 80
DC0, DC1 = 160, 160

_MESH = plsc.VectorSubcoreMesh(
    core_axis_name="c", subcore_axis_name="s", num_cores=NC, num_subcores=NS)


def _worker_ids():
    c = lax.axis_index("c")
    s = lax.axis_index("s")
    return c, s, s * NC + c


def _deg_body(src_hbm, dst_hbm, deg_hbm, idx_v, ones_v, buf_v, deg_sh):
    c, s, wid = _worker_ids()
    for i in range(EC // 16):
        ones_v[pl.ds(i * 16, 16)] = jnp.ones((16,), jnp.float32)
    for i in range(RPT // 16):
        buf_v[pl.ds(i * 16, 16)] = jnp.zeros((16,), jnp.float32)

    pltpu.sync_copy(buf_v, deg_sh.at[pl.ds(s * RPT, RPT)])
    base = jnp.where(c == 0, s * CPW0, NS * CPW0 + s * CPW1)
    dc = 2 * jnp.where(c == 0, CPW0, CPW1)

    @pl.when(c == 0)
    def _():
        pltpu.sync_copy(src_hbm.at[pl.ds(base, CPW0)], idx_v.at[pl.ds(0, CPW0)])
        pltpu.sync_copy(dst_hbm.at[pl.ds(base, CPW0)],
                        idx_v.at[pl.ds(CPW0, CPW0)])

    @pl.when(c != 0)
    def _():
        pltpu.sync_copy(src_hbm.at[pl.ds(base, CPW1)], idx_v.at[pl.ds(0, CPW1)])
        pltpu.sync_copy(dst_hbm.at[pl.ds(base, CPW1)],
                        idx_v.at[pl.ds(CPW1, CPW1)])

    plsc.subcore_barrier()

    def chunk(j, carry):
        pltpu.sync_copy(ones_v, deg_sh.at[idx_v.at[j]], add=True)
        return carry

    lax.fori_loop(0, dc, chunk, 0)
    plsc.subcore_barrier()
    pltpu.sync_copy(deg_sh.at[pl.ds(s * RPT, RPT)], buf_v)
    pltpu.sync_copy(buf_v, deg_hbm.at[pl.ds(c * NP + s * RPT, RPT)])


_deg_kernel = pl.kernel(
    _deg_body,
    out_type=jax.ShapeDtypeStruct((NC * NP,), jnp.float32),
    mesh=_MESH,
    scratch_types=[
        pltpu.VMEM((2 * max(CPW0, CPW1), EC), jnp.int32),
        pltpu.VMEM((EC,), jnp.float32),
        pltpu.VMEM((RPT,), jnp.float32),
        pltpu.VMEM_SHARED((NP,), jnp.float32),
    ],
)


IB = 8
NB0 = CPW0 // IB
NB1 = CPW1 // IB


def _msg_body(d, g_hbm, src_hbm, dst_hbm, agg_hbm, srcv, dstv,
              rows_a, rows_b, sem_a, sem_b, agg_sh):
    c, s, wid = _worker_ids()
    base = jnp.where(c == 0, s * CPW0, NS * CPW0 + s * CPW1)
    nb = jnp.where(c == 0, NB0, NB1)

    def zrow(r, carry):
        for k in range(d // 16):
            rows_a[r, pl.ds(k * 16, 16)] = jnp.zeros((16,), jnp.float32)
        return carry

    lax.fori_loop(0, EC, zrow, 0)

    for t in range(RPT // EC):
        pltpu.sync_copy(rows_a, agg_sh.at[pl.ds(s * RPT + t * EC, EC)])
    @pl.when(nb > 0)
    def _():
        pltpu.sync_copy(src_hbm.at[pl.ds(base, IB)], srcv.at[0])
        pltpu.sync_copy(dst_hbm.at[pl.ds(base, IB)], dstv.at[0])

    plsc.subcore_barrier()

    @pl.when(nb > 0)
    def _():
        pltpu.async_copy(g_hbm.at[srcv.at[0, 0]], rows_a, sem_a)

    def block_body(b, carry):
        sb = lax.rem(b, 2)
        nsb = 1 - sb

        @pl.when(b < nb - 1)
        def _():
            pltpu.sync_copy(src_hbm.at[pl.ds(base + (b + 1) * IB, IB)],
                            srcv.at[nsb])
            pltpu.sync_copy(dst_hbm.at[pl.ds(base + (b + 1) * IB, IB)],
                            dstv.at[nsb])

        for jj in range(IB):
            if jj % 2 == 0:
                buf, sem, obuf, osem = rows_a, sem_a, rows_b, sem_b
            else:
                buf, sem, obuf, osem = rows_b, sem_b, rows_a, sem_a
            if jj < IB - 1:
                pltpu.async_copy(g_hbm.at[srcv.at[sb, jj + 1]], obuf, osem)
            else:
                @pl.when(b < nb - 1)
                def _(nsb=nsb, obuf=obuf, osem=osem):
                    pltpu.async_copy(g_hbm.at[srcv.at[nsb, 0]], obuf, osem)
            pltpu.make_async_copy(g_hbm.at[srcv.at[sb, jj]], buf, sem).wait()
            pltpu.sync_copy(buf, agg_sh.at[dstv.at[sb, jj]], add=True)
        return carry

    lax.fori_loop(0, nb, block_body, 0)
    plsc.subcore_barrier()
    for t in range(RPT // EC):
        tbase = s * RPT + t * EC
        pltpu.sync_copy(agg_sh.at[pl.ds(tbase, EC)], rows_a)
        pltpu.sync_copy(rows_a, agg_hbm.at[c, pl.ds(tbase, EC)])


def _make_msg_kernel(d):
    return pl.kernel(
        functools.partial(_msg_body, d),
        out_type=jax.ShapeDtypeStruct((NC, NP, d), jnp.float32),
        mesh=_MESH,
        scratch_types=[
            pltpu.VMEM((2, IB, EC), jnp.int32),
            pltpu.VMEM((2, IB, EC), jnp.int32),
            pltpu.VMEM((EC, d), jnp.float32),
            pltpu.VMEM((EC, d), jnp.float32),
            pltpu.SemaphoreType.DMA,
            pltpu.SemaphoreType.DMA,
            pltpu.VMEM_SHARED((NP, d), jnp.float32),
        ],
    )


_msg_kernel_128 = _make_msg_kernel(D_HID)


BR = 256
GRID = NP // BR


def _dinv_from(deg_blk):
    deg = deg_blk[0, :] + deg_blk[1, :]
    return 1.0 / jnp.sqrt(jnp.maximum(deg, 1.0))


def _eye(n):
    return (lax.broadcasted_iota(jnp.int32, (n, n), 0)
            == lax.broadcasted_iota(jnp.int32, (n, n), 1)).astype(jnp.float32)


def _stage_a1_body(x_ref, w_ref, h0_ref, gram_ref):
    row = pl.program_id(0) * BR + lax.broadcasted_iota(jnp.int32, (BR, 1), 0)
    xm = jnp.where(row < N, x_ref[...], 0.0)
    h0 = jnp.dot(xm, w_ref[...], preferred_element_type=jnp.float32)
    h0_ref[...] = h0

    @pl.when(pl.program_id(0) == 0)
    def _():
        gram_ref[...] = jnp.zeros_like(gram_ref)

    gram_ref[...] += lax.dot_general(
        h0, h0, (((0,), (0,)), ((), ())), preferred_element_type=jnp.float32)


_stage_a1 = pl.pallas_call(
    _stage_a1_body,
    grid=(GRID,),
    in_specs=[
        pl.BlockSpec((BR, D_IN), lambda i: (i, 0)),
        pl.BlockSpec((D_IN, D_HID), lambda i: (0, 0)),
    ],
    out_specs=[
        pl.BlockSpec((BR, D_HID), lambda i: (i, 0)),
        pl.BlockSpec((D_HID, D_HID), lambda i: (0, 0)),
    ],
    out_shape=[
        jax.ShapeDtypeStruct((NP, D_HID), jnp.float32),
        jax.ShapeDtypeStruct((D_HID, D_HID), jnp.float32),
    ],
)


def _stage_a2_body(h0_ref, deg_ref, g0_ref):
    g0_ref[...] = h0_ref[...] * _dinv_from(deg_ref[...])[:, None]


_stage_a2 = pl.pallas_call(
    _stage_a2_body,
    grid=(GRID,),
    in_specs=[
        pl.BlockSpec((BR, D_HID), lambda i: (i, 0)),
        pl.BlockSpec((NC, BR), lambda i: (0, i)),
    ],
    out_specs=pl.BlockSpec((BR, D_HID), lambda i: (i, 0)),
    out_shape=jax.ShapeDtypeStruct((NP, D_HID), jnp.float32),
)


def _stage_b1_body(h0_ref, gram_ref, lin_ref):
    h0 = h0_ref[...]
    G = gram_ref[...] - _eye(D_HID)
    lin_ref[...] = (1.0 - GL1) * h0 - GL2 * jnp.dot(
        h0, G, preferred_element_type=jnp.float32)


_stage_b1 = pl.pallas_call(
    _stage_b1_body,
    grid=(GRID,),
    in_specs=[
        pl.BlockSpec((BR, D_HID), lambda i: (i, 0)),
        pl.BlockSpec((D_HID, D_HID), lambda i: (0, 0)),
    ],
    out_specs=pl.BlockSpec((BR, D_HID), lambda i: (i, 0)),
    out_shape=jax.ShapeDtypeStruct((NP, D_HID), jnp.float32),
)


def _stage_b2_body(lin_ref, agg_ref, deg_ref, w2_ref, h1_ref, g1_ref):
    dinv = _dinv_from(deg_ref[...])
    aggs = (agg_ref[0] + agg_ref[1]) * dinv[:, None]
    out1 = lin_ref[...] + GL1 * aggs
    h1 = jnp.dot(jnp.maximum(out1, 0.0), w2_ref[...],
                 preferred_element_type=jnp.float32)
    h1_ref[...] = h1
    g1_ref[...] = h1 * dinv[:, None]


_stage_b2 = pl.pallas_call(
    _stage_b2_body,
    grid=(GRID,),
    in_specs=[
        pl.BlockSpec((BR, D_HID), lambda i: (i, 0)),
        pl.BlockSpec((NC, BR, D_HID), lambda i: (0, i, 0)),
        pl.BlockSpec((NC, BR), lambda i: (0, i)),
        pl.BlockSpec((D_HID, D2P), lambda i: (0, 0)),
    ],
    out_specs=[
        pl.BlockSpec((BR, D2P), lambda i: (i, 0)),
        pl.BlockSpec((BR, D2P), lambda i: (i, 0)),
    ],
    out_shape=[
        jax.ShapeDtypeStruct((NP, D2P), jnp.float32),
        jax.ShapeDtypeStruct((NP, D2P), jnp.float32),
    ],
)


def _stage_b3_body(h1_ref, gram1_ref):
    h1 = h1_ref[...]

    @pl.when(pl.program_id(0) == 0)
    def _():
        gram1_ref[...] = jnp.zeros_like(gram1_ref)

    gram1_ref[...] += lax.dot_general(
        h1, h1, (((0,), (0,)), ((), ())), preferred_element_type=jnp.float32)


_stage_b3 = pl.pallas_call(
    _stage_b3_body,
    grid=(GRID,),
    in_specs=[pl.BlockSpec((BR, D2P), lambda i: (i, 0))],
    out_specs=pl.BlockSpec((D2P, D2P), lambda i: (0, 0)),
    out_shape=jax.ShapeDtypeStruct((D2P, D2P), jnp.float32),
)


def _stage_c_body(h1_ref, agg_ref, deg_ref, gram1_ref, out_ref):
    h1 = h1_ref[...]
    dinv = _dinv_from(deg_ref[...])
    aggs = (agg_ref[0] + agg_ref[1]) * dinv[:, None]
    G = gram1_ref[...] - _eye(D2P)
    out2 = ((1.0 - GL1) * h1 + GL1 * aggs
            - GL2 * jnp.dot(h1, G, preferred_element_type=jnp.float32))
    valid = lax.broadcasted_iota(jnp.int32, (BR, D2P), 1) < D_OUT
    masked = jnp.where(valid, out2, -jnp.inf)
    m = jnp.max(masked, axis=1, keepdims=True)
    ex = jnp.where(valid, jnp.exp(out2 - m), 0.0)
    lse = jnp.log(jnp.sum(ex, axis=1, keepdims=True))
    res = out2 - m - lse
    out_ref[...] = res[:, :D_OUT]


_stage_c = pl.pallas_call(
    _stage_c_body,
    grid=(GRID,),
    in_specs=[
        pl.BlockSpec((BR, D2P), lambda i: (i, 0)),
        pl.BlockSpec((NC, BR, D2P), lambda i: (0, i, 0)),
        pl.BlockSpec((NC, BR), lambda i: (0, i)),
        pl.BlockSpec((D2P, D2P), lambda i: (0, 0)),
    ],
    out_specs=pl.BlockSpec((BR, D_OUT), lambda i: (i, 0)),
    out_shape=jax.ShapeDtypeStruct((NP, D_OUT), jnp.float32),
)


def kernel(x, y, edge_index, W_in, W_out):
    del y
    pad = N + (jnp.arange(EP - E, dtype=jnp.int32) % (NP - N))
    srcp = jnp.concatenate([edge_index[0], pad])
    dstp = jnp.concatenate([edge_index[1], pad])
    src2 = srcp.reshape(NW * CPW, EC)
    dst2 = dstp.reshape(NW * CPW, EC)
    w2_pad = jnp.pad(W_out, ((0, 0), (0, D2P - D_OUT)))

    deg = _deg_kernel(src2, dst2).reshape(NC, NP)
    h0, gram0 = _stage_a1(x, W_in)
    g0 = _stage_a2(h0, deg)
    agg1 = _msg_kernel_128(g0, src2, dst2)
    lin = _stage_b1(h0, gram0)
    h1, g1 = _stage_b2(lin, agg1, deg, w2_pad)
    gram1 = _stage_b3(h1)
    agg2 = _msg_kernel_128(g1, src2, dst2)
    return _stage_c(h1, agg2, deg, gram1)[:N]


# speedup vs baseline: 2.9532x; 1.0668x over previous
import functools

import jax
import jax.numpy as jnp
import numpy as np
from jax import lax
from jax.experimental import pallas as pl
from jax.experimental.pallas import tpu as pltpu
from jax.experimental.pallas import tpu_sc as plsc

N = 10000
E = 320000
D_IN = 128
D_HID = 128
D_OUT = 40
GL1 = 0.5
GL2 = 0.05

NC, NS = 2, 16
NW = NC * NS
NP = 10240
RPT = NP // NS
D2P = 64
EC = 128
CPW = 80
EPW = CPW * EC
EP = EPW * NW
DC = 2 * CPW
CPW0, CPW1 = 80, 80
DC0, DC1 = 160, 160

_MESH = plsc.VectorSubcoreMesh(
    core_axis_name="c", subcore_axis_name="s", num_cores=NC, num_subcores=NS)

_PAD_IDX = np.asarray(N + np.arange(EP - E) % (NP - N), np.int32)


def _worker_ids():
    c = lax.axis_index("c")
    s = lax.axis_index("s")
    return c, s, s * NC + c


def _deg_body(src_hbm, dst_hbm, deg_hbm, idx_v, ones_v, buf_v, deg_sh):
    c, s, wid = _worker_ids()
    for i in range(EC // 16):
        ones_v[pl.ds(i * 16, 16)] = jnp.ones((16,), jnp.float32)
    for i in range(RPT // 16):
        buf_v[pl.ds(i * 16, 16)] = jnp.zeros((16,), jnp.float32)

    pltpu.sync_copy(buf_v, deg_sh.at[pl.ds(s * RPT, RPT)])
    base = jnp.where(c == 0, s * CPW0, NS * CPW0 + s * CPW1)
    dc = 2 * jnp.where(c == 0, CPW0, CPW1)

    @pl.when(c == 0)
    def _():
        pltpu.sync_copy(src_hbm.at[pl.ds(base, CPW0)], idx_v.at[pl.ds(0, CPW0)])
        pltpu.sync_copy(dst_hbm.at[pl.ds(base, CPW0)],
                        idx_v.at[pl.ds(CPW0, CPW0)])

    @pl.when(c != 0)
    def _():
        pltpu.sync_copy(src_hbm.at[pl.ds(base, CPW1)], idx_v.at[pl.ds(0, CPW1)])
        pltpu.sync_copy(dst_hbm.at[pl.ds(base, CPW1)],
                        idx_v.at[pl.ds(CPW1, CPW1)])

    plsc.subcore_barrier()

    def chunk(j, carry):
        pltpu.sync_copy(ones_v, deg_sh.at[idx_v.at[j]], add=True)
        return carry

    lax.fori_loop(0, dc, chunk, 0)
    plsc.subcore_barrier()
    pltpu.sync_copy(deg_sh.at[pl.ds(s * RPT, RPT)], buf_v)
    pltpu.sync_copy(buf_v, deg_hbm.at[pl.ds(c * NP + s * RPT, RPT)])


_deg_kernel = pl.kernel(
    _deg_body,
    out_type=jax.ShapeDtypeStruct((NC * NP,), jnp.float32),
    mesh=_MESH,
    scratch_types=[
        pltpu.VMEM((2 * max(CPW0, CPW1), EC), jnp.int32),
        pltpu.VMEM((EC,), jnp.float32),
        pltpu.VMEM((RPT,), jnp.float32),
        pltpu.VMEM_SHARED((NP,), jnp.float32),
    ],
)


IB = 8
NB0 = CPW0 // IB
NB1 = CPW1 // IB


def _msg_body(d, g_hbm, src_hbm, dst_hbm, agg_hbm, srcv, dstv,
              rows_a, rows_b, sem_a, sem_b, agg_sh):
    c, s, wid = _worker_ids()
    base = jnp.where(c == 0, s * CPW0, NS * CPW0 + s * CPW1)
    nb = jnp.where(c == 0, NB0, NB1)

    def zrow(r, carry):
        for k in range(d // 16):
            rows_a[r, pl.ds(k * 16, 16)] = jnp.zeros((16,), jnp.float32)
        return carry

    lax.fori_loop(0, EC, zrow, 0)

    for t in range(RPT // EC):
        pltpu.sync_copy(rows_a, agg_sh.at[pl.ds(s * RPT + t * EC, EC)])
    @pl.when(nb > 0)
    def _():
        pltpu.sync_copy(src_hbm.at[pl.ds(base, IB)], srcv.at[0])
        pltpu.sync_copy(dst_hbm.at[pl.ds(base, IB)], dstv.at[0])

    plsc.subcore_barrier()

    @pl.when(nb > 0)
    def _():
        pltpu.async_copy(g_hbm.at[srcv.at[0, 0]], rows_a, sem_a)

    def block_body(b, carry):
        sb = lax.rem(b, 2)
        nsb = 1 - sb

        @pl.when(b < nb - 1)
        def _():
            pltpu.sync_copy(src_hbm.at[pl.ds(base + (b + 1) * IB, IB)],
                            srcv.at[nsb])
            pltpu.sync_copy(dst_hbm.at[pl.ds(base + (b + 1) * IB, IB)],
                            dstv.at[nsb])

        for jj in range(IB):
            if jj % 2 == 0:
                buf, sem, obuf, osem = rows_a, sem_a, rows_b, sem_b
            else:
                buf, sem, obuf, osem = rows_b, sem_b, rows_a, sem_a
            if jj < IB - 1:
                pltpu.async_copy(g_hbm.at[srcv.at[sb, jj + 1]], obuf, osem)
            else:
                @pl.when(b < nb - 1)
                def _(nsb=nsb, obuf=obuf, osem=osem):
                    pltpu.async_copy(g_hbm.at[srcv.at[nsb, 0]], obuf, osem)
            pltpu.make_async_copy(g_hbm.at[srcv.at[sb, jj]], buf, sem).wait()
            pltpu.sync_copy(buf, agg_sh.at[dstv.at[sb, jj]], add=True)
        return carry

    lax.fori_loop(0, nb, block_body, 0)
    plsc.subcore_barrier()
    for t in range(RPT // EC):
        tbase = s * RPT + t * EC
        pltpu.sync_copy(agg_sh.at[pl.ds(tbase, EC)], rows_a)
        pltpu.sync_copy(rows_a, agg_hbm.at[c, pl.ds(tbase, EC)])


def _make_msg_kernel(d, tc_tiling=True):
    return pl.kernel(
        functools.partial(_msg_body, d),
        out_type=jax.ShapeDtypeStruct((NC, NP, d), jnp.float32),
        mesh=_MESH,
        compiler_params=pltpu.CompilerParams(use_tc_tiling_on_sc=tc_tiling),
        scratch_types=[
            pltpu.VMEM((2, IB, EC), jnp.int32),
            pltpu.VMEM((2, IB, EC), jnp.int32),
            pltpu.VMEM((EC, d), jnp.float32),
            pltpu.VMEM((EC, d), jnp.float32),
            pltpu.SemaphoreType.DMA,
            pltpu.SemaphoreType.DMA,
            pltpu.VMEM_SHARED((NP, d), jnp.float32),
        ],
    )


_msg_kernel_128 = _make_msg_kernel(D_HID)
_msg_kernel_64 = _make_msg_kernel(D2P, tc_tiling=False)


BR = 256
GRID = NP // BR


def _dinv_from(deg_blk):
    deg = deg_blk[0, :] + deg_blk[1, :]
    return 1.0 / jnp.sqrt(jnp.maximum(deg, 1.0))


def _eye(n):
    return (lax.broadcasted_iota(jnp.int32, (n, n), 0)
            == lax.broadcasted_iota(jnp.int32, (n, n), 1)).astype(jnp.float32)


def _stage_a1_body(x_ref, w_ref, h0_ref, gram_ref):
    row = pl.program_id(0) * BR + lax.broadcasted_iota(jnp.int32, (BR, 1), 0)
    xm = jnp.where(row < N, x_ref[...], 0.0)
    h0 = jnp.dot(xm, w_ref[...], preferred_element_type=jnp.float32)
    h0_ref[...] = h0

    @pl.when(pl.program_id(0) == 0)
    def _():
        gram_ref[...] = jnp.zeros_like(gram_ref)

    gram_ref[...] += lax.dot_general(
        h0, h0, (((0,), (0,)), ((), ())), preferred_element_type=jnp.float32)


_stage_a1 = pl.pallas_call(
    _stage_a1_body,
    grid=(GRID,),
    in_specs=[
        pl.BlockSpec((BR, D_IN), lambda i: (i, 0)),
        pl.BlockSpec((D_IN, D_HID), lambda i: (0, 0)),
    ],
    out_specs=[
        pl.BlockSpec((BR, D_HID), lambda i: (i, 0)),
        pl.BlockSpec((D_HID, D_HID), lambda i: (0, 0)),
    ],
    out_shape=[
        jax.ShapeDtypeStruct((NP, D_HID), jnp.float32),
        jax.ShapeDtypeStruct((D_HID, D_HID), jnp.float32),
    ],
)


def _stage_a2_body(h0_ref, deg_ref, g0_ref):
    g0_ref[...] = h0_ref[...] * _dinv_from(deg_ref[...])[:, None]


_stage_a2 = pl.pallas_call(
    _stage_a2_body,
    grid=(GRID,),
    in_specs=[
        pl.BlockSpec((BR, D_HID), lambda i: (i, 0)),
        pl.BlockSpec((NC, BR), lambda i: (0, i)),
    ],
    out_specs=pl.BlockSpec((BR, D_HID), lambda i: (i, 0)),
    out_shape=jax.ShapeDtypeStruct((NP, D_HID), jnp.float32),
)


def _stage_b1_body(h0_ref, gram_ref, lin_ref):
    h0 = h0_ref[...]
    G = gram_ref[...] - _eye(D_HID)
    lin_ref[...] = (1.0 - GL1) * h0 - GL2 * jnp.dot(
        h0, G, preferred_element_type=jnp.float32)


_stage_b1 = pl.pallas_call(
    _stage_b1_body,
    grid=(GRID,),
    in_specs=[
        pl.BlockSpec((BR, D_HID), lambda i: (i, 0)),
        pl.BlockSpec((D_HID, D_HID), lambda i: (0, 0)),
    ],
    out_specs=pl.BlockSpec((BR, D_HID), lambda i: (i, 0)),
    out_shape=jax.ShapeDtypeStruct((NP, D_HID), jnp.float32),
)


def _stage_b2_body(lin_ref, agg_ref, deg_ref, w2_ref, h1_ref, g1_ref):
    dinv = _dinv_from(deg_ref[...])
    aggs = (agg_ref[0] + agg_ref[1]) * dinv[:, None]
    out1 = lin_ref[...] + GL1 * aggs
    h1 = jnp.dot(jnp.maximum(out1, 0.0), w2_ref[...],
                 preferred_element_type=jnp.float32)
    h1_ref[...] = h1
    g1_ref[...] = h1 * dinv[:, None]


_stage_b2 = pl.pallas_call(
    _stage_b2_body,
    grid=(GRID,),
    in_specs=[
        pl.BlockSpec((BR, D_HID), lambda i: (i, 0)),
        pl.BlockSpec((NC, BR, D_HID), lambda i: (0, i, 0)),
        pl.BlockSpec((NC, BR), lambda i: (0, i)),
        pl.BlockSpec((D_HID, D2P), lambda i: (0, 0)),
    ],
    out_specs=[
        pl.BlockSpec((BR, D2P), lambda i: (i, 0)),
        pl.BlockSpec((BR, D2P), lambda i: (i, 0)),
    ],
    out_shape=[
        jax.ShapeDtypeStruct((NP, D2P), jnp.float32),
        jax.ShapeDtypeStruct((NP, D2P), jnp.float32),
    ],
)


def _stage_b3_body(h1_ref, gram1_ref):
    h1 = h1_ref[...]

    @pl.when(pl.program_id(0) == 0)
    def _():
        gram1_ref[...] = jnp.zeros_like(gram1_ref)

    gram1_ref[...] += lax.dot_general(
        h1, h1, (((0,), (0,)), ((), ())), preferred_element_type=jnp.float32)


_stage_b3 = pl.pallas_call(
    _stage_b3_body,
    grid=(GRID,),
    in_specs=[pl.BlockSpec((BR, D2P), lambda i: (i, 0))],
    out_specs=pl.BlockSpec((D2P, D2P), lambda i: (0, 0)),
    out_shape=jax.ShapeDtypeStruct((D2P, D2P), jnp.float32),
)


def _stage_c_body(h1_ref, agg_ref, deg_ref, gram1_ref, out_ref):
    h1 = h1_ref[...]
    dinv = _dinv_from(deg_ref[...])
    aggs = (agg_ref[0] + agg_ref[1]) * dinv[:, None]
    G = gram1_ref[...] - _eye(D2P)
    out2 = ((1.0 - GL1) * h1 + GL1 * aggs
            - GL2 * jnp.dot(h1, G, preferred_element_type=jnp.float32))
    valid = lax.broadcasted_iota(jnp.int32, (BR, D2P), 1) < D_OUT
    masked = jnp.where(valid, out2, -jnp.inf)
    m = jnp.max(masked, axis=1, keepdims=True)
    ex = jnp.where(valid, jnp.exp(out2 - m), 0.0)
    lse = jnp.log(jnp.sum(ex, axis=1, keepdims=True))
    res = out2 - m - lse
    out_ref[...] = res[:, :D_OUT]


_stage_c = pl.pallas_call(
    _stage_c_body,
    grid=(GRID,),
    in_specs=[
        pl.BlockSpec((BR, D2P), lambda i: (i, 0)),
        pl.BlockSpec((NC, BR, D2P), lambda i: (0, i, 0)),
        pl.BlockSpec((NC, BR), lambda i: (0, i)),
        pl.BlockSpec((D2P, D2P), lambda i: (0, 0)),
    ],
    out_specs=pl.BlockSpec((BR, D_OUT), lambda i: (i, 0)),
    out_shape=jax.ShapeDtypeStruct((NP, D_OUT), jnp.float32),
)


def kernel(x, y, edge_index, W_in, W_out):
    del y
    pad = jnp.asarray(_PAD_IDX)
    srcp = jnp.concatenate([edge_index[0], pad])
    dstp = jnp.concatenate([edge_index[1], pad])
    src2 = srcp.reshape(NW * CPW, EC)
    dst2 = dstp.reshape(NW * CPW, EC)
    w2_pad = jnp.pad(W_out, ((0, 0), (0, D2P - D_OUT)))

    deg = _deg_kernel(src2, dst2).reshape(NC, NP)
    h0, gram0 = _stage_a1(x, W_in)
    g0 = _stage_a2(h0, deg)
    agg1 = _msg_kernel_128(g0, src2, dst2)
    lin = _stage_b1(h0, gram0)
    h1, g1 = _stage_b2(lin, agg1, deg, w2_pad)
    gram1 = _stage_b3(h1)
    agg2 = _msg_kernel_64(g1, src2, dst2)
    return _stage_c(h1, agg2, deg, gram1)[:N]


# speedup vs baseline: 3.3435x; 1.1321x over previous
import functools

import jax
import jax.numpy as jnp
import numpy as np
from jax import lax
from jax.experimental import pallas as pl
from jax.experimental.pallas import tpu as pltpu
from jax.experimental.pallas import tpu_sc as plsc

N = 10000
E = 320000
D_IN = 128
D_HID = 128
D_OUT = 40
GL1 = 0.5
GL2 = 0.05

NC, NS = 2, 16
NW = NC * NS
NP = 10240
RPT = NP // NS
D2P = 64
EC = 128
CPW = 80
EPW = CPW * EC
EP = EPW * NW
DC = 2 * CPW
CPW0, CPW1 = 80, 80
DC0, DC1 = 160, 160

_MESH = plsc.VectorSubcoreMesh(
    core_axis_name="c", subcore_axis_name="s", num_cores=NC, num_subcores=NS)

_PAD_IDX = np.asarray(N + np.arange(EP - E) % (NP - N), np.int32)


def _worker_ids():
    c = lax.axis_index("c")
    s = lax.axis_index("s")
    return c, s, s * NC + c


def _deg_body(src_hbm, dst_hbm, deg_hbm, idx_v, ones_v, buf_v, deg_sh):
    c, s, wid = _worker_ids()
    for i in range(EC // 16):
        ones_v[pl.ds(i * 16, 16)] = jnp.ones((16,), jnp.float32)
    for i in range(RPT // 16):
        buf_v[pl.ds(i * 16, 16)] = jnp.zeros((16,), jnp.float32)

    pltpu.sync_copy(buf_v, deg_sh.at[pl.ds(s * RPT, RPT)])
    base = jnp.where(c == 0, s * CPW0, NS * CPW0 + s * CPW1)
    dc = 2 * jnp.where(c == 0, CPW0, CPW1)

    @pl.when(c == 0)
    def _():
        pltpu.sync_copy(src_hbm.at[pl.ds(base, CPW0)], idx_v.at[pl.ds(0, CPW0)])
        pltpu.sync_copy(dst_hbm.at[pl.ds(base, CPW0)],
                        idx_v.at[pl.ds(CPW0, CPW0)])

    @pl.when(c != 0)
    def _():
        pltpu.sync_copy(src_hbm.at[pl.ds(base, CPW1)], idx_v.at[pl.ds(0, CPW1)])
        pltpu.sync_copy(dst_hbm.at[pl.ds(base, CPW1)],
                        idx_v.at[pl.ds(CPW1, CPW1)])

    plsc.subcore_barrier()

    def chunk(j, carry):
        pltpu.sync_copy(ones_v, deg_sh.at[idx_v.at[j]], add=True)
        return carry

    lax.fori_loop(0, dc, chunk, 0)
    plsc.subcore_barrier()
    pltpu.sync_copy(deg_sh.at[pl.ds(s * RPT, RPT)], buf_v)
    pltpu.sync_copy(buf_v, deg_hbm.at[pl.ds(c * NP + s * RPT, RPT)])


_deg_kernel = pl.kernel(
    _deg_body,
    out_type=jax.ShapeDtypeStruct((NC * NP,), jnp.float32),
    mesh=_MESH,
    scratch_types=[
        pltpu.VMEM((2 * max(CPW0, CPW1), EC), jnp.int32),
        pltpu.VMEM((EC,), jnp.float32),
        pltpu.VMEM((RPT,), jnp.float32),
        pltpu.VMEM_SHARED((NP,), jnp.float32),
    ],
)


IB = 8
NB0 = CPW0 // IB
NB1 = CPW1 // IB


def _msg_body(d, g_hbm, src_hbm, dst_hbm, agg_hbm, srcv, dstv,
              rows_a, rows_b, sem_a, sem_b, agg_sh):
    c, s, wid = _worker_ids()
    base = jnp.where(c == 0, s * CPW0, NS * CPW0 + s * CPW1)
    nb = jnp.where(c == 0, NB0, NB1)

    def zrow(r, carry):
        for k in range(d // 16):
            rows_a[r, pl.ds(k * 16, 16)] = jnp.zeros((16,), jnp.float32)
        return carry

    lax.fori_loop(0, EC, zrow, 0)

    for t in range(RPT // EC):
        pltpu.sync_copy(rows_a, agg_sh.at[pl.ds(s * RPT + t * EC, EC)])
    @pl.when(nb > 0)
    def _():
        pltpu.sync_copy(src_hbm.at[pl.ds(base, IB)], srcv.at[0])
        pltpu.sync_copy(dst_hbm.at[pl.ds(base, IB)], dstv.at[0])

    plsc.subcore_barrier()

    @pl.when(nb > 0)
    def _():
        pltpu.async_copy(g_hbm.at[srcv.at[0, 0]], rows_a, sem_a)

    def block_body(b, carry):
        sb = lax.rem(b, 2)
        nsb = 1 - sb

        @pl.when(b < nb - 1)
        def _():
            pltpu.sync_copy(src_hbm.at[pl.ds(base + (b + 1) * IB, IB)],
                            srcv.at[nsb])
            pltpu.sync_copy(dst_hbm.at[pl.ds(base + (b + 1) * IB, IB)],
                            dstv.at[nsb])

        for jj in range(IB):
            if jj % 2 == 0:
                buf, sem, obuf, osem = rows_a, sem_a, rows_b, sem_b
            else:
                buf, sem, obuf, osem = rows_b, sem_b, rows_a, sem_a
            if jj < IB - 1:
                pltpu.async_copy(g_hbm.at[srcv.at[sb, jj + 1]], obuf, osem)
            else:
                @pl.when(b < nb - 1)
                def _(nsb=nsb, obuf=obuf, osem=osem):
                    pltpu.async_copy(g_hbm.at[srcv.at[nsb, 0]], obuf, osem)
            pltpu.make_async_copy(g_hbm.at[srcv.at[sb, jj]], buf, sem).wait()
            pltpu.sync_copy(buf, agg_sh.at[dstv.at[sb, jj]], add=True)
        return carry

    lax.fori_loop(0, nb, block_body, 0)
    plsc.subcore_barrier()
    for t in range(RPT // EC):
        tbase = s * RPT + t * EC
        pltpu.sync_copy(agg_sh.at[pl.ds(tbase, EC)], rows_a)
        pltpu.sync_copy(rows_a, agg_hbm.at[c, pl.ds(tbase, EC)])


def _make_msg_kernel(d, tc_tiling=True):
    return pl.kernel(
        functools.partial(_msg_body, d),
        out_type=jax.ShapeDtypeStruct((NC, NP, d), jnp.float32),
        mesh=_MESH,
        compiler_params=pltpu.CompilerParams(use_tc_tiling_on_sc=tc_tiling),
        scratch_types=[
            pltpu.VMEM((2, IB, EC), jnp.int32),
            pltpu.VMEM((2, IB, EC), jnp.int32),
            pltpu.VMEM((EC, d), jnp.float32),
            pltpu.VMEM((EC, d), jnp.float32),
            pltpu.SemaphoreType.DMA,
            pltpu.SemaphoreType.DMA,
            pltpu.VMEM_SHARED((NP, d), jnp.float32),
        ],
    )


_msg_kernel_128 = _make_msg_kernel(D_HID)
_msg_kernel_64 = _make_msg_kernel(D2P, tc_tiling=False)


BR = 512
GRID = NP // BR


def _dinv_from(deg_blk):
    deg = deg_blk[0, :] + deg_blk[1, :]
    return 1.0 / jnp.sqrt(jnp.maximum(deg, 1.0))


def _eye(n):
    return (lax.broadcasted_iota(jnp.int32, (n, n), 0)
            == lax.broadcasted_iota(jnp.int32, (n, n), 1)).astype(jnp.float32)


def _stage_a1_body(x_ref, w_ref, h0_ref, gram_ref):
    row = pl.program_id(0) * BR + lax.broadcasted_iota(jnp.int32, (BR, 1), 0)
    xm = jnp.where(row < N, x_ref[...], 0.0)
    h0 = jnp.dot(xm, w_ref[...], preferred_element_type=jnp.float32)
    h0_ref[...] = h0

    @pl.when(pl.program_id(0) == 0)
    def _():
        gram_ref[...] = jnp.zeros_like(gram_ref)

    gram_ref[...] += lax.dot_general(
        h0, h0, (((0,), (0,)), ((), ())), preferred_element_type=jnp.float32)


_stage_a1 = pl.pallas_call(
    _stage_a1_body,
    grid=(GRID,),
    in_specs=[
        pl.BlockSpec((BR, D_IN), lambda i: (i, 0)),
        pl.BlockSpec((D_IN, D_HID), lambda i: (0, 0)),
    ],
    out_specs=[
        pl.BlockSpec((BR, D_HID), lambda i: (i, 0)),
        pl.BlockSpec((D_HID, D_HID), lambda i: (0, 0)),
    ],
    out_shape=[
        jax.ShapeDtypeStruct((NP, D_HID), jnp.float32),
        jax.ShapeDtypeStruct((D_HID, D_HID), jnp.float32),
    ],
)


def _stage_a2_body(h0_ref, deg_ref, g0_ref):
    g0_ref[...] = h0_ref[...] * _dinv_from(deg_ref[...])[:, None]


_stage_a2 = pl.pallas_call(
    _stage_a2_body,
    grid=(GRID,),
    in_specs=[
        pl.BlockSpec((BR, D_HID), lambda i: (i, 0)),
        pl.BlockSpec((NC, BR), lambda i: (0, i)),
    ],
    out_specs=pl.BlockSpec((BR, D_HID), lambda i: (i, 0)),
    out_shape=jax.ShapeDtypeStruct((NP, D_HID), jnp.float32),
)


def _stage_b1_body(h0_ref, gram_ref, lin_ref):
    h0 = h0_ref[...]
    G = gram_ref[...] - _eye(D_HID)
    lin_ref[...] = (1.0 - GL1) * h0 - GL2 * jnp.dot(
        h0, G, preferred_element_type=jnp.float32)


_stage_b1 = pl.pallas_call(
    _stage_b1_body,
    grid=(GRID,),
    in_specs=[
        pl.BlockSpec((BR, D_HID), lambda i: (i, 0)),
        pl.BlockSpec((D_HID, D_HID), lambda i: (0, 0)),
    ],
    out_specs=pl.BlockSpec((BR, D_HID), lambda i: (i, 0)),
    out_shape=jax.ShapeDtypeStruct((NP, D_HID), jnp.float32),
)


def _stage_b2_body(lin_ref, agg_ref, deg_ref, w2_ref, h1_ref, g1_ref):
    dinv = _dinv_from(deg_ref[...])
    aggs = (agg_ref[0] + agg_ref[1]) * dinv[:, None]
    out1 = lin_ref[...] + GL1 * aggs
    h1 = jnp.dot(jnp.maximum(out1, 0.0), w2_ref[...],
                 preferred_element_type=jnp.float32)
    h1_ref[...] = h1
    g1_ref[...] = h1 * dinv[:, None]


_stage_b2 = pl.pallas_call(
    _stage_b2_body,
    grid=(GRID,),
    in_specs=[
        pl.BlockSpec((BR, D_HID), lambda i: (i, 0)),
        pl.BlockSpec((NC, BR, D_HID), lambda i: (0, i, 0)),
        pl.BlockSpec((NC, BR), lambda i: (0, i)),
        pl.BlockSpec((D_HID, D2P), lambda i: (0, 0)),
    ],
    out_specs=[
        pl.BlockSpec((BR, D2P), lambda i: (i, 0)),
        pl.BlockSpec((BR, D2P), lambda i: (i, 0)),
    ],
    out_shape=[
        jax.ShapeDtypeStruct((NP, D2P), jnp.float32),
        jax.ShapeDtypeStruct((NP, D2P), jnp.float32),
    ],
)


def _stage_b3_body(h1_ref, gram1_ref):
    h1 = h1_ref[...]

    @pl.when(pl.program_id(0) == 0)
    def _():
        gram1_ref[...] = jnp.zeros_like(gram1_ref)

    gram1_ref[...] += lax.dot_general(
        h1, h1, (((0,), (0,)), ((), ())), preferred_element_type=jnp.float32)


_stage_b3 = pl.pallas_call(
    _stage_b3_body,
    grid=(GRID,),
    in_specs=[pl.BlockSpec((BR, D2P), lambda i: (i, 0))],
    out_specs=pl.BlockSpec((D2P, D2P), lambda i: (0, 0)),
    out_shape=jax.ShapeDtypeStruct((D2P, D2P), jnp.float32),
)


def _stage_c_body(h1_ref, agg_ref, deg_ref, gram1_ref, out_ref):
    h1 = h1_ref[...]
    dinv = _dinv_from(deg_ref[...])
    aggs = (agg_ref[0] + agg_ref[1]) * dinv[:, None]
    G = gram1_ref[...] - _eye(D2P)
    out2 = ((1.0 - GL1) * h1 + GL1 * aggs
            - GL2 * jnp.dot(h1, G, preferred_element_type=jnp.float32))
    valid = lax.broadcasted_iota(jnp.int32, (BR, D2P), 1) < D_OUT
    masked = jnp.where(valid, out2, -jnp.inf)
    m = jnp.max(masked, axis=1, keepdims=True)
    ex = jnp.where(valid, jnp.exp(out2 - m), 0.0)
    lse = jnp.log(jnp.sum(ex, axis=1, keepdims=True))
    res = out2 - m - lse
    out_ref[...] = res[:, :D_OUT]


_stage_c = pl.pallas_call(
    _stage_c_body,
    grid=(GRID,),
    in_specs=[
        pl.BlockSpec((BR, D2P), lambda i: (i, 0)),
        pl.BlockSpec((NC, BR, D2P), lambda i: (0, i, 0)),
        pl.BlockSpec((NC, BR), lambda i: (0, i)),
        pl.BlockSpec((D2P, D2P), lambda i: (0, 0)),
    ],
    out_specs=pl.BlockSpec((BR, D_OUT), lambda i: (i, 0)),
    out_shape=jax.ShapeDtypeStruct((NP, D_OUT), jnp.float32),
)


def kernel(x, y, edge_index, W_in, W_out):
    del y
    pad = jnp.asarray(_PAD_IDX)
    srcp = jnp.concatenate([edge_index[0], pad])
    dstp = jnp.concatenate([edge_index[1], pad])
    src2 = srcp.reshape(NW * CPW, EC)
    dst2 = dstp.reshape(NW * CPW, EC)
    w2_pad = jnp.pad(W_out, ((0, 0), (0, D2P - D_OUT)))

    deg = _deg_kernel(src2, dst2).reshape(NC, NP)
    h0, gram0 = _stage_a1(x, W_in)
    g0 = _stage_a2(h0, deg)
    agg1 = _msg_kernel_128(g0, src2, dst2)
    lin = _stage_b1(h0, gram0)
    h1, g1 = _stage_b2(lin, agg1, deg, w2_pad)
    gram1 = _stage_b3(h1)
    agg2 = _msg_kernel_64(g1, src2, dst2)
    return _stage_c(h1, agg2, deg, gram1)[:N]


# speedup vs baseline: 3.5120x; 1.0504x over previous
import functools

import jax
import jax.numpy as jnp
import numpy as np
from jax import lax
from jax.experimental import pallas as pl
from jax.experimental.pallas import tpu as pltpu
from jax.experimental.pallas import tpu_sc as plsc

N = 10000
E = 320000
D_IN = 128
D_HID = 128
D_OUT = 40
GL1 = 0.5
GL2 = 0.05

NC, NS = 2, 16
NW = NC * NS
NP = 10240
RPT = NP // NS
D2P = 64
EC = 128
CPW = 80
EPW = CPW * EC
EP = EPW * NW
DC = 2 * CPW
CPW0, CPW1 = 80, 80
DC0, DC1 = 160, 160

_MESH = plsc.VectorSubcoreMesh(
    core_axis_name="c", subcore_axis_name="s", num_cores=NC, num_subcores=NS)

_PAD_IDX = np.asarray(N + np.arange(EP - E) % (NP - N), np.int32)


def _worker_ids():
    c = lax.axis_index("c")
    s = lax.axis_index("s")
    return c, s, s * NC + c


def _deg_body(src_hbm, dst_hbm, deg_hbm, idx_v, ones_v, buf_v, deg_sh):
    c, s, wid = _worker_ids()
    for i in range(EC // 16):
        ones_v[pl.ds(i * 16, 16)] = jnp.ones((16,), jnp.float32)
    for i in range(RPT // 16):
        buf_v[pl.ds(i * 16, 16)] = jnp.zeros((16,), jnp.float32)

    pltpu.sync_copy(buf_v, deg_sh.at[pl.ds(s * RPT, RPT)])
    base = jnp.where(c == 0, s * CPW0, NS * CPW0 + s * CPW1)
    dc = 2 * jnp.where(c == 0, CPW0, CPW1)

    @pl.when(c == 0)
    def _():
        pltpu.sync_copy(src_hbm.at[pl.ds(base, CPW0)], idx_v.at[pl.ds(0, CPW0)])
        pltpu.sync_copy(dst_hbm.at[pl.ds(base, CPW0)],
                        idx_v.at[pl.ds(CPW0, CPW0)])

    @pl.when(c != 0)
    def _():
        pltpu.sync_copy(src_hbm.at[pl.ds(base, CPW1)], idx_v.at[pl.ds(0, CPW1)])
        pltpu.sync_copy(dst_hbm.at[pl.ds(base, CPW1)],
                        idx_v.at[pl.ds(CPW1, CPW1)])

    plsc.subcore_barrier()

    def chunk(j, carry):
        pltpu.sync_copy(ones_v, deg_sh.at[idx_v.at[j]], add=True)
        return carry

    lax.fori_loop(0, dc, chunk, 0)
    plsc.subcore_barrier()
    pltpu.sync_copy(deg_sh.at[pl.ds(s * RPT, RPT)], buf_v)
    pltpu.sync_copy(buf_v, deg_hbm.at[pl.ds(c * NP + s * RPT, RPT)])


_deg_kernel = pl.kernel(
    _deg_body,
    out_type=jax.ShapeDtypeStruct((NC * NP,), jnp.float32),
    mesh=_MESH,
    scratch_types=[
        pltpu.VMEM((2 * max(CPW0, CPW1), EC), jnp.int32),
        pltpu.VMEM((EC,), jnp.float32),
        pltpu.VMEM((RPT,), jnp.float32),
        pltpu.VMEM_SHARED((NP,), jnp.float32),
    ],
)


IB = 8
NB0 = CPW0 // IB
NB1 = CPW1 // IB


def _msg_body(d, g_hbm, src_hbm, dst_hbm, agg_hbm, srcv, dstv,
              rows_a, rows_b, sem_a, sem_b, agg_sh):
    c, s, wid = _worker_ids()
    base = jnp.where(c == 0, s * CPW0, NS * CPW0 + s * CPW1)
    nb = jnp.where(c == 0, NB0, NB1)

    def zrow(r, carry):
        for k in range(d // 16):
            rows_a[r, pl.ds(k * 16, 16)] = jnp.zeros((16,), jnp.float32)
        return carry

    lax.fori_loop(0, EC, zrow, 0)

    for t in range(RPT // EC):
        pltpu.sync_copy(rows_a, agg_sh.at[pl.ds(s * RPT + t * EC, EC)])
    @pl.when(nb > 0)
    def _():
        pltpu.sync_copy(src_hbm.at[pl.ds(base, IB)], srcv.at[0])
        pltpu.sync_copy(dst_hbm.at[pl.ds(base, IB)], dstv.at[0])

    plsc.subcore_barrier()

    @pl.when(nb > 0)
    def _():
        pltpu.async_copy(g_hbm.at[srcv.at[0, 0]], rows_a, sem_a)

    def block_body(b, carry):
        sb = lax.rem(b, 2)
        nsb = 1 - sb

        @pl.when(b < nb - 1)
        def _():
            pltpu.sync_copy(src_hbm.at[pl.ds(base + (b + 1) * IB, IB)],
                            srcv.at[nsb])
            pltpu.sync_copy(dst_hbm.at[pl.ds(base + (b + 1) * IB, IB)],
                            dstv.at[nsb])

        for jj in range(IB):
            if jj % 2 == 0:
                buf, sem, obuf, osem = rows_a, sem_a, rows_b, sem_b
            else:
                buf, sem, obuf, osem = rows_b, sem_b, rows_a, sem_a
            if jj < IB - 1:
                pltpu.async_copy(g_hbm.at[srcv.at[sb, jj + 1]], obuf, osem)
            else:
                @pl.when(b < nb - 1)
                def _(nsb=nsb, obuf=obuf, osem=osem):
                    pltpu.async_copy(g_hbm.at[srcv.at[nsb, 0]], obuf, osem)
            pltpu.make_async_copy(g_hbm.at[srcv.at[sb, jj]], buf, sem).wait()
            pltpu.sync_copy(buf, agg_sh.at[dstv.at[sb, jj]], add=True)
        return carry

    lax.fori_loop(0, nb, block_body, 0)
    plsc.subcore_barrier()
    for t in range(RPT // EC):
        tbase = s * RPT + t * EC
        pltpu.sync_copy(agg_sh.at[pl.ds(tbase, EC)], rows_a)
        pltpu.sync_copy(rows_a, agg_hbm.at[c, pl.ds(tbase, EC)])


def _make_msg_kernel(d, tc_tiling=True):
    return pl.kernel(
        functools.partial(_msg_body, d),
        out_type=jax.ShapeDtypeStruct((NC, NP, d), jnp.float32),
        mesh=_MESH,
        compiler_params=pltpu.CompilerParams(use_tc_tiling_on_sc=tc_tiling),
        scratch_types=[
            pltpu.VMEM((2, IB, EC), jnp.int32),
            pltpu.VMEM((2, IB, EC), jnp.int32),
            pltpu.VMEM((EC, d), jnp.float32),
            pltpu.VMEM((EC, d), jnp.float32),
            pltpu.SemaphoreType.DMA,
            pltpu.SemaphoreType.DMA,
            pltpu.VMEM_SHARED((NP, d), jnp.float32),
        ],
    )


_msg_kernel_128 = _make_msg_kernel(D_HID)
_msg_kernel_64 = _make_msg_kernel(D2P, tc_tiling=False)


BR = 1024
GRID = NP // BR


def _dinv_from(deg_blk):
    deg = deg_blk[0, :] + deg_blk[1, :]
    return 1.0 / jnp.sqrt(jnp.maximum(deg, 1.0))


def _eye(n):
    return (lax.broadcasted_iota(jnp.int32, (n, n), 0)
            == lax.broadcasted_iota(jnp.int32, (n, n), 1)).astype(jnp.float32)


def _stage_a1_body(x_ref, w_ref, h0_ref, gram_ref):
    row = pl.program_id(0) * BR + lax.broadcasted_iota(jnp.int32, (BR, 1), 0)
    xm = jnp.where(row < N, x_ref[...], 0.0)
    h0 = jnp.dot(xm, w_ref[...], preferred_element_type=jnp.float32)
    h0_ref[...] = h0

    @pl.when(pl.program_id(0) == 0)
    def _():
        gram_ref[...] = jnp.zeros_like(gram_ref)

    gram_ref[...] += lax.dot_general(
        h0, h0, (((0,), (0,)), ((), ())), preferred_element_type=jnp.float32)


_stage_a1 = pl.pallas_call(
    _stage_a1_body,
    grid=(GRID,),
    in_specs=[
        pl.BlockSpec((BR, D_IN), lambda i: (i, 0)),
        pl.BlockSpec((D_IN, D_HID), lambda i: (0, 0)),
    ],
    out_specs=[
        pl.BlockSpec((BR, D_HID), lambda i: (i, 0)),
        pl.BlockSpec((D_HID, D_HID), lambda i: (0, 0)),
    ],
    out_shape=[
        jax.ShapeDtypeStruct((NP, D_HID), jnp.float32),
        jax.ShapeDtypeStruct((D_HID, D_HID), jnp.float32),
    ],
)


def _stage_a2_body(h0_ref, deg_ref, g0_ref):
    g0_ref[...] = h0_ref[...] * _dinv_from(deg_ref[...])[:, None]


_stage_a2 = pl.pallas_call(
    _stage_a2_body,
    grid=(GRID,),
    in_specs=[
        pl.BlockSpec((BR, D_HID), lambda i: (i, 0)),
        pl.BlockSpec((NC, BR), lambda i: (0, i)),
    ],
    out_specs=pl.BlockSpec((BR, D_HID), lambda i: (i, 0)),
    out_shape=jax.ShapeDtypeStruct((NP, D_HID), jnp.float32),
)


def _stage_b1_body(h0_ref, gram_ref, lin_ref):
    h0 = h0_ref[...]
    G = gram_ref[...] - _eye(D_HID)
    lin_ref[...] = (1.0 - GL1) * h0 - GL2 * jnp.dot(
        h0, G, preferred_element_type=jnp.float32)


_stage_b1 = pl.pallas_call(
    _stage_b1_body,
    grid=(GRID,),
    in_specs=[
        pl.BlockSpec((BR, D_HID), lambda i: (i, 0)),
        pl.BlockSpec((D_HID, D_HID), lambda i: (0, 0)),
    ],
    out_specs=pl.BlockSpec((BR, D_HID), lambda i: (i, 0)),
    out_shape=jax.ShapeDtypeStruct((NP, D_HID), jnp.float32),
)


def _stage_b2_body(lin_ref, agg_ref, deg_ref, w2_ref, h1_ref, g1_ref):
    dinv = _dinv_from(deg_ref[...])
    aggs = (agg_ref[0] + agg_ref[1]) * dinv[:, None]
    out1 = lin_ref[...] + GL1 * aggs
    h1 = jnp.dot(jnp.maximum(out1, 0.0), w2_ref[...],
                 preferred_element_type=jnp.float32)
    h1_ref[...] = h1
    g1_ref[...] = h1 * dinv[:, None]


_stage_b2 = pl.pallas_call(
    _stage_b2_body,
    grid=(GRID,),
    in_specs=[
        pl.BlockSpec((BR, D_HID), lambda i: (i, 0)),
        pl.BlockSpec((NC, BR, D_HID), lambda i: (0, i, 0)),
        pl.BlockSpec((NC, BR), lambda i: (0, i)),
        pl.BlockSpec((D_HID, D2P), lambda i: (0, 0)),
    ],
    out_specs=[
        pl.BlockSpec((BR, D2P), lambda i: (i, 0)),
        pl.BlockSpec((BR, D2P), lambda i: (i, 0)),
    ],
    out_shape=[
        jax.ShapeDtypeStruct((NP, D2P), jnp.float32),
        jax.ShapeDtypeStruct((NP, D2P), jnp.float32),
    ],
)


def _stage_b3_body(h1_ref, gram1_ref):
    h1 = h1_ref[...]

    @pl.when(pl.program_id(0) == 0)
    def _():
        gram1_ref[...] = jnp.zeros_like(gram1_ref)

    gram1_ref[...] += lax.dot_general(
        h1, h1, (((0,), (0,)), ((), ())), preferred_element_type=jnp.float32)


_stage_b3 = pl.pallas_call(
    _stage_b3_body,
    grid=(GRID,),
    in_specs=[pl.BlockSpec((BR, D2P), lambda i: (i, 0))],
    out_specs=pl.BlockSpec((D2P, D2P), lambda i: (0, 0)),
    out_shape=jax.ShapeDtypeStruct((D2P, D2P), jnp.float32),
)


def _stage_c_body(h1_ref, agg_ref, deg_ref, gram1_ref, out_ref):
    h1 = h1_ref[...]
    dinv = _dinv_from(deg_ref[...])
    aggs = (agg_ref[0] + agg_ref[1]) * dinv[:, None]
    G = gram1_ref[...] - _eye(D2P)
    out2 = ((1.0 - GL1) * h1 + GL1 * aggs
            - GL2 * jnp.dot(h1, G, preferred_element_type=jnp.float32))
    valid = lax.broadcasted_iota(jnp.int32, (BR, D2P), 1) < D_OUT
    masked = jnp.where(valid, out2, -jnp.inf)
    m = jnp.max(masked, axis=1, keepdims=True)
    ex = jnp.where(valid, jnp.exp(out2 - m), 0.0)
    lse = jnp.log(jnp.sum(ex, axis=1, keepdims=True))
    res = out2 - m - lse
    out_ref[...] = res[:, :D_OUT]


_stage_c = pl.pallas_call(
    _stage_c_body,
    grid=(GRID,),
    in_specs=[
        pl.BlockSpec((BR, D2P), lambda i: (i, 0)),
        pl.BlockSpec((NC, BR, D2P), lambda i: (0, i, 0)),
        pl.BlockSpec((NC, BR), lambda i: (0, i)),
        pl.BlockSpec((D2P, D2P), lambda i: (0, 0)),
    ],
    out_specs=pl.BlockSpec((BR, D_OUT), lambda i: (i, 0)),
    out_shape=jax.ShapeDtypeStruct((NP, D_OUT), jnp.float32),
)


def kernel(x, y, edge_index, W_in, W_out):
    del y
    pad = jnp.asarray(_PAD_IDX)
    srcp = jnp.concatenate([edge_index[0], pad])
    dstp = jnp.concatenate([edge_index[1], pad])
    src2 = srcp.reshape(NW * CPW, EC)
    dst2 = dstp.reshape(NW * CPW, EC)
    w2_pad = jnp.pad(W_out, ((0, 0), (0, D2P - D_OUT)))

    deg = _deg_kernel(src2, dst2).reshape(NC, NP)
    h0, gram0 = _stage_a1(x, W_in)
    g0 = _stage_a2(h0, deg)
    agg1 = _msg_kernel_128(g0, src2, dst2)
    lin = _stage_b1(h0, gram0)
    h1, g1 = _stage_b2(lin, agg1, deg, w2_pad)
    gram1 = _stage_b3(h1)
    agg2 = _msg_kernel_64(g1, src2, dst2)
    return _stage_c(h1, agg2, deg, gram1)[:N]
